# baseline (speedup 1.0000x reference)
import functools

import jax
import jax.numpy as jnp
from jax.experimental import pallas as pl

_NUM_EMBEDDINGS = 1024
_EMBEDDING_DIM = 64
_COMMITMENT_COST = 0.25


def _vq_block(x_ref, w_ref, idx_ref, qst_ref, loss_ref):
    nb = x_ref.shape[0]
    c = x_ref.shape[1]
    hw = x_ref.shape[2] * x_ref.shape[3]
    w = w_ref[...]
    xs = [x_ref[i].reshape(c, hw) for i in range(nb)]
    f = jnp.concatenate([x.T for x in xs], axis=0)
    fs = jnp.sum(f * f, axis=1, keepdims=True)
    s = jax.lax.dot_general(
        f, w, (((1,), (1,)), ((), ())),
        preferred_element_type=jnp.float32)
    ws = jnp.sum(w * w, axis=1)[None, :]
    d = fs - 2.0 * s + ws
    minval = jnp.min(d, axis=1, keepdims=True)
    jidx = jax.lax.broadcasted_iota(jnp.int32, d.shape, 1).astype(jnp.float32)
    idx_f = jnp.min(jnp.where(d == minval, jidx, 2048.0),
                    axis=1, keepdims=True)
    idx_ref[...] = idx_f.astype(jnp.int32)
    onehot = (jidx == idx_f).astype(jnp.bfloat16)
    q = jax.lax.dot_general(
        w.astype(jnp.bfloat16), onehot, (((0,), (1,)), ((), ())),
        preferred_element_type=jnp.float32)
    part = jnp.float32(0.0)
    for i in range(nb):
        xi = xs[i]
        qi = q[:, i * hw:(i + 1) * hw]
        qst_ref[i] = (xi + (qi - xi)).reshape(qst_ref.shape[1:])
        part += jnp.sum((qi - xi) ** 2)
    @pl.when(pl.program_id(0) == 0)
    def _init():
        loss_ref[...] = jnp.zeros_like(loss_ref)
    loss_ref[...] += part[None, None]


@functools.partial(jax.jit, static_argnames=())
def kernel(inputs, W):
    b, c, h, w = inputs.shape
    hw = h * w
    n = b * hw
    nb = 2
    idx2, qst4, loss_sum = pl.pallas_call(
        _vq_block,
        grid=(b // nb,),
        in_specs=[
            pl.BlockSpec((nb, c, h, w), lambda i: (i, 0, 0, 0)),
            pl.BlockSpec((_NUM_EMBEDDINGS, c), lambda i: (0, 0)),
        ],
        out_specs=[
            pl.BlockSpec((nb * hw, 1), lambda i: (i, 0)),
            pl.BlockSpec((nb, c, h, w), lambda i: (i, 0, 0, 0)),
            pl.BlockSpec((1, 1), lambda i: (0, 0)),
        ],
        out_shape=[
            jax.ShapeDtypeStruct((n, 1), jnp.int32),
            jax.ShapeDtypeStruct((b, c, h, w), jnp.float32),
            jax.ShapeDtypeStruct((1, 1), jnp.float32),
        ],
    )(inputs, W)
    discrete = idx2.reshape(b, h, w)
    quantized_out = qst4
    m = loss_sum[0, 0] / n / c
    loss = m + _COMMITMENT_COST * m
    return (discrete, quantized_out, loss)


# speedup vs baseline: 1.5742x; 1.5742x over previous
import functools

import jax
import jax.numpy as jnp
from jax.experimental import pallas as pl

_NUM_EMBEDDINGS = 1024
_EMBEDDING_DIM = 64
_COMMITMENT_COST = 0.25


def _vq_block(f_ref, w_ref, idx_ref, qst_ref, loss_ref):
    f = f_ref[...]
    w = w_ref[...]
    fs = jnp.sum(f * f, axis=1, keepdims=True)
    s = jax.lax.dot_general(
        f, w, (((1,), (1,)), ((), ())),
        preferred_element_type=jnp.float32)
    ws = jnp.sum(w * w, axis=1)[None, :]
    d = fs - 2.0 * s + ws
    minval = jnp.min(d, axis=1, keepdims=True)
    jidx = jax.lax.broadcasted_iota(jnp.int32, d.shape, 1).astype(jnp.float32)
    idx_f = jnp.min(jnp.where(d == minval, jidx, 2048.0),
                    axis=1, keepdims=True)
    idx_ref[...] = idx_f.astype(jnp.int32)
    onehot = (jidx == idx_f).astype(jnp.bfloat16)
    q = jnp.dot(onehot, w.astype(jnp.bfloat16),
                preferred_element_type=jnp.float32)
    qst_ref[...] = f + (q - f)
    part = jnp.sum((q - f) ** 2)
    @pl.when(pl.program_id(0) == 0)
    def _init():
        loss_ref[...] = jnp.zeros_like(loss_ref)
    loss_ref[...] += part[None, None]


@functools.partial(jax.jit, static_argnames=())
def kernel(inputs, W):
    b, c, h, w = inputs.shape
    n = b * h * w
    flat = jnp.transpose(inputs, (0, 2, 3, 1)).reshape(n, c)
    blk = 4096
    grid = n // blk
    idx2, qst, loss_sum = pl.pallas_call(
        _vq_block,
        grid=(grid,),
        in_specs=[
            pl.BlockSpec((blk, c), lambda i: (i, 0)),
            pl.BlockSpec((_NUM_EMBEDDINGS, c), lambda i: (0, 0)),
        ],
        out_specs=[
            pl.BlockSpec((blk, 1), lambda i: (i, 0)),
            pl.BlockSpec((blk, c), lambda i: (i, 0)),
            pl.BlockSpec((1, 1), lambda i: (0, 0)),
        ],
        out_shape=[
            jax.ShapeDtypeStruct((n, 1), jnp.int32),
            jax.ShapeDtypeStruct((n, c), jnp.float32),
            jax.ShapeDtypeStruct((1, 1), jnp.float32),
        ],
    )(flat, W)
    discrete = idx2.reshape(b, h, w)
    quantized_out = jnp.transpose(qst.reshape(b, h, w, c), (0, 3, 1, 2))
    m = loss_sum[0, 0] / n / c
    loss = m + _COMMITMENT_COST * m
    return (discrete, quantized_out, loss)
